# baseline (speedup 1.0000x reference)
import functools

import jax
import jax.numpy as jnp
from jax import lax
from jax.experimental import pallas as pl
from jax.experimental.pallas import tpu as pltpu
from jax.experimental.pallas import tpu_sc as plsc

B = 4
L = 8192
D = 64
N_IDX = B * L

_info = plsc.get_sparse_core_info()
NC, NS = _info.num_cores, _info.num_subcores
NW = NC * NS
B_W = N_IDX // NW
CH = 128
NCH = B_W // CH

_mesh = plsc.VectorSubcoreMesh(core_axis_name="c", subcore_axis_name="s")


@functools.partial(
    pl.kernel,
    mesh=_mesh,
    compiler_params=pltpu.CompilerParams(use_tc_tiling_on_sc=True),
    out_type=jax.ShapeDtypeStruct((N_IDX, D), jnp.float32),
    scratch_types=[
        pltpu.VMEM((NCH, CH), jnp.int32),
        pltpu.VMEM((NCH, CH), jnp.int32),
        pltpu.VMEM((2, CH, 2 * D), jnp.float32),
        pltpu.VMEM((2, CH, D), jnp.float32),
        pltpu.SemaphoreType.DMA,
        pltpu.SemaphoreType.DMA,
        pltpu.SemaphoreType.DMA,
        pltpu.SemaphoreType.DMA,
    ],
)
def _embed_gather(idx2_hbm, par_hbm, table2_hbm, out_hbm,
                  idx2_v, par_v, buf_v, sel_v, g_sem0, g_sem1,
                  o_sem0, o_sem1):
    wid = lax.axis_index("s") * NC + lax.axis_index("c")
    base = wid * B_W
    pltpu.sync_copy(idx2_hbm.at[wid], idx2_v)
    pltpu.sync_copy(par_hbm.at[wid], par_v)

    g_sems = (g_sem0, g_sem1)
    o_sems = (o_sem0, o_sem1)

    def gather_copy(j):
        return pltpu.make_async_copy(
            table2_hbm.at[idx2_v.at[j]], buf_v.at[j % 2], g_sems[j % 2]
        )

    def out_copy(j):
        return pltpu.make_async_copy(
            sel_v.at[j % 2],
            out_hbm.at[pl.ds(base + j * CH, CH)],
            o_sems[j % 2],
        )

    gather_copy(0).start()
    for j in range(NCH):
        if j + 1 < NCH:
            gather_copy(j + 1).start()
        gather_copy(j).wait()
        if j >= 2:
            out_copy(j - 2).wait()
        jb = j % 2

        def select_group(g, _):
            p16 = par_v[j, pl.ds(g * 16, 16)]
            for l in range(16):
                i = g * 16 + l
                off = p16[l] * D
                for c0 in range(0, D, 16):
                    sel_v[jb, i, pl.ds(c0, 16)] = (
                        buf_v[jb, i, pl.ds(off + c0, 16)]
                    )
            return _

        lax.fori_loop(0, CH // 16, select_group, 0)
        out_copy(j).start()

    out_copy(NCH - 2).wait()
    out_copy(NCH - 1).wait()


def kernel(x_BL, table):
    idx = x_BL.reshape(NW, NCH, CH).astype(jnp.int32)
    idx2 = idx >> 1
    par = idx & 1
    table2 = table.reshape(500000, 2 * D)
    out = _embed_gather(idx2, par, table2)
    return out.reshape(B, L, D)


# speedup vs baseline: 1.7099x; 1.7099x over previous
import functools

import jax
import jax.numpy as jnp
from jax import lax
from jax.experimental import pallas as pl
from jax.experimental.pallas import tpu as pltpu
from jax.experimental.pallas import tpu_sc as plsc

B = 4
L = 8192
D = 64
N_IDX = B * L

_info = plsc.get_sparse_core_info()
NC, NS = _info.num_cores, _info.num_subcores
NW = NC * NS
B_W = N_IDX // NW
CH = 128
NCH = B_W // CH
NG = CH // 16

_mesh = plsc.VectorSubcoreMesh(core_axis_name="c", subcore_axis_name="s")


@functools.partial(
    pl.kernel,
    mesh=_mesh,
    compiler_params=pltpu.CompilerParams(use_tc_tiling_on_sc=True),
    out_type=jax.ShapeDtypeStruct((N_IDX, D), jnp.float32),
    scratch_types=[
        pltpu.VMEM((B_W,), jnp.int32),
        pltpu.VMEM((3, CH, D), jnp.float32),
        pltpu.SemaphoreType.DMA,
        pltpu.SemaphoreType.DMA,
        pltpu.SemaphoreType.DMA,
        pltpu.SemaphoreType.DMA,
    ],
)
def _embed_gather(idx_hbm, table_hbm, out_hbm, idx_v, buf_v,
                  g_sem, o_sem0, o_sem1, o_sem2):
    wid = lax.axis_index("s") * NC + lax.axis_index("c")
    base = wid * B_W
    b = wid // (L // B_W)
    l0 = (wid % (L // B_W)) * B_W
    pltpu.sync_copy(idx_hbm.at[b, pl.ds(l0, B_W)], idx_v)

    o_sems = (o_sem0, o_sem1, o_sem2)

    def issue_chunk(j):
        jb = j % 3

        def issue_group(g, _):
            v16 = idx_v[pl.ds(j * CH + g * 16, 16)]
            for l in range(16):
                pltpu.make_async_copy(
                    table_hbm.at[pl.ds(v16[l], 1)],
                    buf_v.at[jb, pl.ds(g * 16 + l, 1)],
                    g_sem,
                ).start()
            return _

        lax.fori_loop(0, NG, issue_group, 0)

    def wait_chunk(j):
        def wait_group(g, _):
            for l in range(16):
                pltpu.make_async_copy(
                    table_hbm.at[pl.ds(0, 1)],
                    buf_v.at[0, pl.ds(0, 1)],
                    g_sem,
                ).wait()
            return _

        lax.fori_loop(0, NG, wait_group, 0)

    def out_copy(j):
        return pltpu.make_async_copy(
            buf_v.at[j % 3],
            out_hbm.at[pl.ds(base + j * CH, CH)],
            o_sems[j % 3],
        )

    issue_chunk(0)
    for j in range(NCH):
        if j + 1 < NCH:
            if j >= 2:
                out_copy(j - 2).wait()
            issue_chunk(j + 1)
        wait_chunk(j)
        out_copy(j).start()

    out_copy(NCH - 3).wait()
    out_copy(NCH - 2).wait()
    out_copy(NCH - 1).wait()


def kernel(x_BL, table):
    out = _embed_gather(x_BL.astype(jnp.int32), table)
    return out.reshape(B, L, D)
